# baseline (speedup 1.0000x reference)
import functools

import jax
import jax.numpy as jnp
from jax import lax
from jax.experimental import pallas as pl
from jax.experimental.pallas import tpu as pltpu
from jax.experimental.pallas import tpu_sc as plsc

_N_IN = 64
_N_OUT = 32
_TILE = 8192
_L = 16


def _sc_body(n_edges, n_cores, n_workers, src_hbm, dst_hbm, w_hbm, a_hbm,
             cnt_hbm, src_v, dst_v, w_v, acc_v, row_v, cnt_v):
    wid = lax.axis_index("s") * n_cores + lax.axis_index("c")
    pltpu.sync_copy(src_hbm, src_v)
    pltpu.sync_copy(dst_hbm, dst_v)
    pltpu.sync_copy(w_hbm, w_v)
    lanes = lax.iota(jnp.int32, _L)

    for dst0 in range(0, _N_OUT, n_workers):
        n = wid + dst0

        @pl.when(n < _N_OUT)
        def _one_row():
            for j in range(_L * _N_IN // _L):
                acc_v[pl.ds(j * _L, _L)] = jnp.zeros((_L,), jnp.float32)
            cnt_v[...] = jnp.zeros((_L,), jnp.float32)

            def scan(i, carry):
                s = src_v[pl.ds(i * _L, _L)]
                d = dst_v[pl.ds(i * _L, _L)]
                ww = w_v[pl.ds(i * _L, _L)]
                m_d = d == n
                cnt_v[...] = cnt_v[...] + jnp.where(m_d, 1.0, 0.0)
                m = m_d & (s < _N_IN)
                idx = lanes * _N_IN + jnp.where(m, s, 0)
                plsc.addupdate_scatter(acc_v, [idx], jnp.where(m, ww, 0.0),
                                       mask=m)
                return carry

            lax.fori_loop(0, n_edges // _L, scan, 0)

            for j in range(_N_IN // _L):
                acc = acc_v[pl.ds(j * _L, _L)]
                for l in range(1, _L):
                    acc = acc + acc_v[pl.ds(l * _N_IN + j * _L, _L)]
                row_v[pl.ds(j * _L, _L)] = acc

            pltpu.sync_copy(row_v, a_hbm.at[pl.ds(n * _N_IN, _N_IN)])
            pltpu.sync_copy(cnt_v, cnt_hbm.at[pl.ds(n * _L, _L)])


def _sc_build(src, dst, w):
    e = src.shape[0]
    info = plsc.get_sparse_core_info()
    nw = info.num_cores * info.num_subcores
    body = functools.partial(_sc_body, e, info.num_cores, nw)
    return pl.kernel(
        body,
        out_type=(
            jax.ShapeDtypeStruct((_N_OUT * _N_IN,), jnp.float32),
            jax.ShapeDtypeStruct((_N_OUT * _L,), jnp.float32),
        ),
        mesh=plsc.VectorSubcoreMesh(core_axis_name="c", subcore_axis_name="s"),
        scratch_types=[
            pltpu.VMEM((e,), jnp.int32),
            pltpu.VMEM((e,), jnp.int32),
            pltpu.VMEM((e,), jnp.float32),
            pltpu.VMEM((_L * _N_IN,), jnp.float32),
            pltpu.VMEM((_N_IN,), jnp.float32),
            pltpu.VMEM((_L,), jnp.float32),
        ],
        compiler_params=pltpu.CompilerParams(needs_layout_passes=False),
    )(src, dst, w)


def _tc_body(x_ref, a_ref, cnt_ref, b_ref, r_ref, o_ref):
    m = (jnp.sum(cnt_ref[...], axis=1)[None, :] > 0.0).astype(jnp.float32)
    z = b_ref[...] + r_ref[...] * jnp.dot(
        x_ref[...], a_ref[...].T, preferred_element_type=jnp.float32)
    o_ref[...] = jax.nn.sigmoid(z) * m


def kernel(inputs, weights, biases, responses, src_col, dst_idx):
    b, n_in = inputs.shape
    e = weights.shape[0]
    a_flat, cnt_flat = _sc_build(src_col.astype(jnp.int32),
                                 dst_idx.astype(jnp.int32), weights)
    a = a_flat.reshape(_N_OUT, n_in)
    cnt = cnt_flat.reshape(_N_OUT, _L)
    b2d = biases[:_N_OUT].reshape(1, _N_OUT)
    r2d = responses[:_N_OUT].reshape(1, _N_OUT)

    grid = (b // _TILE,)
    return pl.pallas_call(
        _tc_body,
        grid=grid,
        in_specs=[
            pl.BlockSpec((_TILE, n_in), lambda i: (i, 0)),
            pl.BlockSpec((_N_OUT, n_in), lambda i: (0, 0)),
            pl.BlockSpec((_N_OUT, _L), lambda i: (0, 0)),
            pl.BlockSpec((1, _N_OUT), lambda i: (0, 0)),
            pl.BlockSpec((1, _N_OUT), lambda i: (0, 0)),
        ],
        out_specs=pl.BlockSpec((_TILE, _N_OUT), lambda i: (i, 0)),
        out_shape=jax.ShapeDtypeStruct((b, _N_OUT), inputs.dtype),
    )(inputs, a, cnt, b2d, r2d)


# speedup vs baseline: 1.0356x; 1.0356x over previous
import functools

import jax
import jax.numpy as jnp
from jax import lax
from jax.experimental import pallas as pl
from jax.experimental.pallas import tpu as pltpu
from jax.experimental.pallas import tpu_sc as plsc

_N_IN = 64
_N_OUT = 32
_TILE = 8192
_L = 16


_UNROLL = 4


def _sc_body(n_edges, n_cores, n_workers, src_hbm, dst_hbm, w_hbm, a_hbm,
             cnt_hbm, src_v, dst_v, w_v, acc_v, row_v, cnt_v,
             sem0, sem1, sem2):
    wid = lax.axis_index("s") * n_cores + lax.axis_index("c")
    c0 = pltpu.async_copy(src_hbm, src_v, sem0)
    c1 = pltpu.async_copy(dst_hbm, dst_v, sem1)
    c2 = pltpu.async_copy(w_hbm, w_v, sem2)
    c0.wait()
    c1.wait()
    c2.wait()
    lanes = lax.iota(jnp.int32, _L)

    for dst0 in range(0, _N_OUT, n_workers):
        n = wid + dst0

        @pl.when(n < _N_OUT)
        def _one_row():
            for j in range(_L * _N_IN // _L):
                acc_v[pl.ds(j * _L, _L)] = jnp.zeros((_L,), jnp.float32)

            def scan(i, cnt):
                for u in range(_UNROLL):
                    off = (i * _UNROLL + u) * _L
                    s = src_v[pl.ds(off, _L)]
                    d = dst_v[pl.ds(off, _L)]
                    ww = w_v[pl.ds(off, _L)]
                    m_d = d == n
                    cnt = cnt + jnp.where(m_d, 1.0, 0.0)
                    m = m_d & (s < _N_IN)
                    idx = lanes * _N_IN + jnp.where(m, s, 0)
                    plsc.addupdate_scatter(acc_v, [idx],
                                           jnp.where(m, ww, 0.0), mask=m)
                return cnt

            cnt = lax.fori_loop(0, n_edges // (_L * _UNROLL), scan,
                                jnp.zeros((_L,), jnp.float32))
            cnt_v[...] = cnt

            for j in range(_N_IN // _L):
                acc = acc_v[pl.ds(j * _L, _L)]
                for l in range(1, _L):
                    acc = acc + acc_v[pl.ds(l * _N_IN + j * _L, _L)]
                row_v[pl.ds(j * _L, _L)] = acc

            pltpu.sync_copy(row_v, a_hbm.at[pl.ds(n * _N_IN, _N_IN)])
            pltpu.sync_copy(cnt_v, cnt_hbm.at[pl.ds(n * _L, _L)])


def _sc_build(src, dst, w):
    e = src.shape[0]
    info = plsc.get_sparse_core_info()
    nw = info.num_cores * info.num_subcores
    body = functools.partial(_sc_body, e, info.num_cores, nw)
    return pl.kernel(
        body,
        out_type=(
            jax.ShapeDtypeStruct((_N_OUT * _N_IN,), jnp.float32),
            jax.ShapeDtypeStruct((_N_OUT * _L,), jnp.float32),
        ),
        mesh=plsc.VectorSubcoreMesh(core_axis_name="c", subcore_axis_name="s"),
        scratch_types=[
            pltpu.VMEM((e,), jnp.int32),
            pltpu.VMEM((e,), jnp.int32),
            pltpu.VMEM((e,), jnp.float32),
            pltpu.VMEM((_L * _N_IN,), jnp.float32),
            pltpu.VMEM((_N_IN,), jnp.float32),
            pltpu.VMEM((_L,), jnp.float32),
            pltpu.SemaphoreType.DMA,
            pltpu.SemaphoreType.DMA,
            pltpu.SemaphoreType.DMA,
        ],
        compiler_params=pltpu.CompilerParams(needs_layout_passes=False),
    )(src, dst, w)


def _tc_body(x_ref, a_ref, cnt_ref, b_ref, r_ref, o_ref):
    m = (jnp.sum(cnt_ref[...], axis=1)[None, :] > 0.0).astype(jnp.float32)
    z = b_ref[...] + r_ref[...] * jnp.dot(
        x_ref[...], a_ref[...].T, preferred_element_type=jnp.float32)
    o_ref[...] = jax.nn.sigmoid(z) * m


def kernel(inputs, weights, biases, responses, src_col, dst_idx):
    b, n_in = inputs.shape
    e = weights.shape[0]
    a_flat, cnt_flat = _sc_build(src_col.astype(jnp.int32),
                                 dst_idx.astype(jnp.int32), weights)
    a = a_flat.reshape(_N_OUT, n_in)
    cnt = cnt_flat.reshape(_N_OUT, _L)
    b2d = biases[:_N_OUT].reshape(1, _N_OUT)
    r2d = responses[:_N_OUT].reshape(1, _N_OUT)

    grid = (b // _TILE,)
    return pl.pallas_call(
        _tc_body,
        grid=grid,
        in_specs=[
            pl.BlockSpec((_TILE, n_in), lambda i: (i, 0)),
            pl.BlockSpec((_N_OUT, n_in), lambda i: (0, 0)),
            pl.BlockSpec((_N_OUT, _L), lambda i: (0, 0)),
            pl.BlockSpec((1, _N_OUT), lambda i: (0, 0)),
            pl.BlockSpec((1, _N_OUT), lambda i: (0, 0)),
        ],
        out_specs=pl.BlockSpec((_TILE, _N_OUT), lambda i: (i, 0)),
        out_shape=jax.ShapeDtypeStruct((b, _N_OUT), inputs.dtype),
    )(inputs, a, cnt, b2d, r2d)


# speedup vs baseline: 1.0631x; 1.0265x over previous
import functools

import jax
import jax.numpy as jnp
from jax import lax
from jax.experimental import pallas as pl
from jax.experimental.pallas import tpu as pltpu
from jax.experimental.pallas import tpu_sc as plsc

_N_IN = 64
_N_OUT = 32
_TILE = 8192
_L = 16


_UNROLL = 4


def _sc_body(n_edges, n_cores, n_workers, src_hbm, dst_hbm, w_hbm, ac_hbm,
             src_v, dst_v, w_v, acc_v, row_v, sem0, sem1, sem2):
    wid = lax.axis_index("s") * n_cores + lax.axis_index("c")
    c0 = pltpu.async_copy(src_hbm, src_v, sem0)
    c1 = pltpu.async_copy(dst_hbm, dst_v, sem1)
    c2 = pltpu.async_copy(w_hbm, w_v, sem2)
    c0.wait()
    c1.wait()
    c2.wait()
    lanes = lax.iota(jnp.int32, _L)

    for dst0 in range(0, _N_OUT, n_workers):
        n = wid + dst0

        @pl.when(n < _N_OUT)
        def _one_row():
            for j in range(_L * _N_IN // _L):
                acc_v[pl.ds(j * _L, _L)] = jnp.zeros((_L,), jnp.float32)

            def scan(i, cnt):
                for u in range(_UNROLL):
                    off = (i * _UNROLL + u) * _L
                    s = src_v[pl.ds(off, _L)]
                    d = dst_v[pl.ds(off, _L)]
                    ww = w_v[pl.ds(off, _L)]
                    m_d = d == n
                    cnt = cnt + jnp.where(m_d, 1.0, 0.0)
                    m = m_d & (s < _N_IN)
                    idx = lanes * _N_IN + jnp.where(m, s, 0)
                    plsc.addupdate_scatter(acc_v, [idx],
                                           jnp.where(m, ww, 0.0), mask=m)
                return cnt

            cnt = lax.fori_loop(0, n_edges // (_L * _UNROLL), scan,
                                jnp.zeros((_L,), jnp.float32))

            for j in range(_N_IN // _L):
                acc = acc_v[pl.ds(j * _L, _L)]
                for l in range(1, _L):
                    acc = acc + acc_v[pl.ds(l * _N_IN + j * _L, _L)]
                row_v[pl.ds(j * _L, _L)] = acc
            row_v[pl.ds(_N_IN, _L)] = cnt

            pltpu.sync_copy(
                row_v, ac_hbm.at[pl.ds(n * (_N_IN + _L), _N_IN + _L)])


def _sc_build(src, dst, w):
    e = src.shape[0]
    info = plsc.get_sparse_core_info()
    nw = info.num_cores * info.num_subcores
    body = functools.partial(_sc_body, e, info.num_cores, nw)
    return pl.kernel(
        body,
        out_type=jax.ShapeDtypeStruct((_N_OUT * (_N_IN + _L),), jnp.float32),
        mesh=plsc.VectorSubcoreMesh(core_axis_name="c", subcore_axis_name="s"),
        scratch_types=[
            pltpu.VMEM((e,), jnp.int32),
            pltpu.VMEM((e,), jnp.int32),
            pltpu.VMEM((e,), jnp.float32),
            pltpu.VMEM((_L * _N_IN,), jnp.float32),
            pltpu.VMEM((_N_IN + _L,), jnp.float32),
            pltpu.SemaphoreType.DMA,
            pltpu.SemaphoreType.DMA,
            pltpu.SemaphoreType.DMA,
        ],
        compiler_params=pltpu.CompilerParams(needs_layout_passes=False),
    )(src, dst, w)


def _tc_body(x_ref, ac_ref, b_ref, r_ref, o_ref):
    ac = ac_ref[...]
    a = ac[:, :_N_IN]
    m = (jnp.sum(ac[:, _N_IN:], axis=1)[None, :] > 0.0).astype(jnp.float32)
    z = b_ref[...] + r_ref[...] * jnp.dot(
        x_ref[...], a.T, preferred_element_type=jnp.float32)
    o_ref[...] = jax.nn.sigmoid(z) * m


def kernel(inputs, weights, biases, responses, src_col, dst_idx):
    b, n_in = inputs.shape
    e = weights.shape[0]
    ac_flat = _sc_build(src_col.astype(jnp.int32),
                        dst_idx.astype(jnp.int32), weights)
    ac = ac_flat.reshape(_N_OUT, n_in + _L)
    b2d = biases[:_N_OUT].reshape(1, _N_OUT)
    r2d = responses[:_N_OUT].reshape(1, _N_OUT)

    grid = (b // _TILE,)
    return pl.pallas_call(
        _tc_body,
        grid=grid,
        in_specs=[
            pl.BlockSpec((_TILE, n_in), lambda i: (i, 0)),
            pl.BlockSpec((_N_OUT, n_in + _L), lambda i: (0, 0)),
            pl.BlockSpec((1, _N_OUT), lambda i: (0, 0)),
            pl.BlockSpec((1, _N_OUT), lambda i: (0, 0)),
        ],
        out_specs=pl.BlockSpec((_TILE, _N_OUT), lambda i: (i, 0)),
        out_shape=jax.ShapeDtypeStruct((b, _N_OUT), inputs.dtype),
    )(inputs, ac, b2d, r2d)


# speedup vs baseline: 1.0791x; 1.0150x over previous
import functools

import jax
import jax.numpy as jnp
from jax import lax
from jax.experimental import pallas as pl
from jax.experimental.pallas import tpu as pltpu
from jax.experimental.pallas import tpu_sc as plsc

_N_IN = 64
_N_OUT = 32
_TILE = 8192
_L = 16


_UNROLL = 4


def _sc_body(n_edges, n_cores, n_workers, src_hbm, dst_hbm, w_hbm, ac_hbm,
             src_v, dst_v, w_v, acc_v, row_v, sem0, sem1, sem2):
    wid = lax.axis_index("s") * n_cores + lax.axis_index("c")
    c0 = pltpu.async_copy(src_hbm, src_v, sem0)
    c1 = pltpu.async_copy(dst_hbm, dst_v, sem1)
    c2 = pltpu.async_copy(w_hbm, w_v, sem2)
    c0.wait()
    c1.wait()
    c2.wait()
    lanes = lax.iota(jnp.int32, _L)

    for dst0 in range(0, _N_OUT, n_workers):
        n = wid + dst0

        @pl.when(n < _N_OUT)
        def _one_row():
            for j in range(_L * _N_IN // _L):
                acc_v[pl.ds(j * _L, _L)] = jnp.zeros((_L,), jnp.float32)

            def scan(i, cnt):
                for u in range(_UNROLL):
                    off = (i * _UNROLL + u) * _L
                    s = src_v[pl.ds(off, _L)]
                    d = dst_v[pl.ds(off, _L)]
                    ww = w_v[pl.ds(off, _L)]
                    m_d = d == n
                    cnt = cnt + jnp.where(m_d, 1.0, 0.0)
                    m = m_d & (s < _N_IN)
                    idx = lanes * _N_IN + jnp.where(m, s, 0)
                    plsc.addupdate_scatter(acc_v, [idx],
                                           jnp.where(m, ww, 0.0), mask=m)
                return cnt

            cnt = lax.fori_loop(0, n_edges // (_L * _UNROLL), scan,
                                jnp.zeros((_L,), jnp.float32))

            for j in range(_N_IN // _L):
                acc = acc_v[pl.ds(j * _L, _L)]
                for l in range(1, _L):
                    acc = acc + acc_v[pl.ds(l * _N_IN + j * _L, _L)]
                row_v[pl.ds(j * _L, _L)] = acc
            row_v[pl.ds(_N_IN, _L)] = cnt

            pltpu.sync_copy(
                row_v, ac_hbm.at[pl.ds(n * (_N_IN + _L), _N_IN + _L)])


def _sc_build(src, dst, w):
    e = src.shape[0]
    info = plsc.get_sparse_core_info()
    n_cores = 1
    nw = n_cores * info.num_subcores
    body = functools.partial(_sc_body, e, n_cores, nw)
    return pl.kernel(
        body,
        out_type=jax.ShapeDtypeStruct((_N_OUT * (_N_IN + _L),), jnp.float32),
        mesh=plsc.VectorSubcoreMesh(core_axis_name="c", subcore_axis_name="s",
                                    num_cores=1),
        scratch_types=[
            pltpu.VMEM((e,), jnp.int32),
            pltpu.VMEM((e,), jnp.int32),
            pltpu.VMEM((e,), jnp.float32),
            pltpu.VMEM((_L * _N_IN,), jnp.float32),
            pltpu.VMEM((_N_IN + _L,), jnp.float32),
            pltpu.SemaphoreType.DMA,
            pltpu.SemaphoreType.DMA,
            pltpu.SemaphoreType.DMA,
        ],
        compiler_params=pltpu.CompilerParams(needs_layout_passes=False),
    )(src, dst, w)


def _tc_body(x_ref, ac_ref, b_ref, r_ref, o_ref):
    ac = ac_ref[...]
    a = ac[:, :_N_IN]
    m = (jnp.sum(ac[:, _N_IN:], axis=1)[None, :] > 0.0).astype(jnp.float32)
    z = b_ref[...] + r_ref[...] * jnp.dot(
        x_ref[...], a.T, preferred_element_type=jnp.float32)
    o_ref[...] = jax.nn.sigmoid(z) * m


def kernel(inputs, weights, biases, responses, src_col, dst_idx):
    b, n_in = inputs.shape
    e = weights.shape[0]
    ac_flat = _sc_build(src_col.astype(jnp.int32),
                        dst_idx.astype(jnp.int32), weights)
    ac = ac_flat.reshape(_N_OUT, n_in + _L)
    b2d = biases[:_N_OUT].reshape(1, _N_OUT)
    r2d = responses[:_N_OUT].reshape(1, _N_OUT)

    grid = (b // _TILE,)
    return pl.pallas_call(
        _tc_body,
        grid=grid,
        in_specs=[
            pl.BlockSpec((_TILE, n_in), lambda i: (i, 0)),
            pl.BlockSpec((_N_OUT, n_in + _L), lambda i: (0, 0)),
            pl.BlockSpec((1, _N_OUT), lambda i: (0, 0)),
            pl.BlockSpec((1, _N_OUT), lambda i: (0, 0)),
        ],
        out_specs=pl.BlockSpec((_TILE, _N_OUT), lambda i: (i, 0)),
        out_shape=jax.ShapeDtypeStruct((b, _N_OUT), inputs.dtype),
    )(inputs, ac, b2d, r2d)


# speedup vs baseline: 1.1335x; 1.0505x over previous
import functools

import jax
import jax.numpy as jnp
from jax import lax
from jax.experimental import pallas as pl
from jax.experimental.pallas import tpu as pltpu
from jax.experimental.pallas import tpu_sc as plsc

_N_IN = 64
_N_OUT = 32
_TILE = 8192
_L = 16


_UNROLL = 4


def _sc_body(n_edges, n_cores, n_workers, src_hbm, dst_hbm, w_hbm, ac_hbm,
             src_v, dst_v, w_v, acc_v, row_v, sem0, sem1, sem2):
    wid = lax.axis_index("s") * n_cores + lax.axis_index("c")
    c0 = pltpu.async_copy(src_hbm, src_v, sem0)
    c1 = pltpu.async_copy(dst_hbm, dst_v, sem1)
    c2 = pltpu.async_copy(w_hbm, w_v, sem2)
    c0.wait()
    c1.wait()
    c2.wait()
    lanes = lax.iota(jnp.int32, _L)
    rpw = _N_OUT // n_workers
    n0 = wid * rpw
    width = rpw * _N_IN

    for j in range(_L * width // _L):
        acc_v[pl.ds(j * _L, _L)] = jnp.zeros((_L,), jnp.float32)

    def scan(i, cnts):
        for u in range(_UNROLL):
            off = (i * _UNROLL + u) * _L
            s = src_v[pl.ds(off, _L)]
            d = dst_v[pl.ds(off, _L)]
            ww = w_v[pl.ds(off, _L)]
            rel = d - n0
            m_d = (rel >= 0) & (rel < rpw)
            cnts = tuple(
                c + jnp.where(d == (n0 + r), 1.0, 0.0)
                for r, c in enumerate(cnts))
            m = m_d & (s < _N_IN)
            idx = (lanes * width + jnp.where(m, rel, 0) * _N_IN
                   + jnp.where(m, s, 0))
            plsc.addupdate_scatter(acc_v, [idx], jnp.where(m, ww, 0.0),
                                   mask=m)
        return cnts

    cnts = lax.fori_loop(
        0, n_edges // (_L * _UNROLL), scan,
        tuple(jnp.zeros((_L,), jnp.float32) for _ in range(rpw)))

    for r in range(rpw):
        for j in range(_N_IN // _L):
            acc = acc_v[pl.ds(r * _N_IN + j * _L, _L)]
            for l in range(1, _L):
                acc = acc + acc_v[pl.ds(l * width + r * _N_IN + j * _L, _L)]
            row_v[pl.ds(r * (_N_IN + _L) + j * _L, _L)] = acc
        row_v[pl.ds(r * (_N_IN + _L) + _N_IN, _L)] = cnts[r]

    blk = rpw * (_N_IN + _L)
    pltpu.sync_copy(row_v, ac_hbm.at[pl.ds(wid * blk, blk)])


def _sc_build(src, dst, w):
    e = src.shape[0]
    info = plsc.get_sparse_core_info()
    n_cores = 1
    nw = n_cores * info.num_subcores
    body = functools.partial(_sc_body, e, n_cores, nw)
    return pl.kernel(
        body,
        out_type=jax.ShapeDtypeStruct((_N_OUT * (_N_IN + _L),), jnp.float32),
        mesh=plsc.VectorSubcoreMesh(core_axis_name="c", subcore_axis_name="s",
                                    num_cores=1),
        scratch_types=[
            pltpu.VMEM((e,), jnp.int32),
            pltpu.VMEM((e,), jnp.int32),
            pltpu.VMEM((e,), jnp.float32),
            pltpu.VMEM((_L * (_N_OUT // nw) * _N_IN,), jnp.float32),
            pltpu.VMEM(((_N_OUT // nw) * (_N_IN + _L),), jnp.float32),
            pltpu.SemaphoreType.DMA,
            pltpu.SemaphoreType.DMA,
            pltpu.SemaphoreType.DMA,
        ],
        compiler_params=pltpu.CompilerParams(needs_layout_passes=False),
    )(src, dst, w)


def _tc_body(x_ref, ac_ref, b_ref, r_ref, o_ref):
    ac = ac_ref[...]
    a = ac[:, :_N_IN]
    m = (jnp.sum(ac[:, _N_IN:], axis=1)[None, :] > 0.0).astype(jnp.float32)
    z = b_ref[...] + r_ref[...] * jnp.dot(
        x_ref[...], a.T, preferred_element_type=jnp.float32)
    o_ref[...] = jax.nn.sigmoid(z) * m


def kernel(inputs, weights, biases, responses, src_col, dst_idx):
    b, n_in = inputs.shape
    e = weights.shape[0]
    ac_flat = _sc_build(src_col.astype(jnp.int32),
                        dst_idx.astype(jnp.int32), weights)
    ac = ac_flat.reshape(_N_OUT, n_in + _L)
    b2d = biases[:_N_OUT].reshape(1, _N_OUT)
    r2d = responses[:_N_OUT].reshape(1, _N_OUT)

    grid = (b // _TILE,)
    return pl.pallas_call(
        _tc_body,
        grid=grid,
        in_specs=[
            pl.BlockSpec((_TILE, n_in), lambda i: (i, 0)),
            pl.BlockSpec((_N_OUT, n_in + _L), lambda i: (0, 0)),
            pl.BlockSpec((1, _N_OUT), lambda i: (0, 0)),
            pl.BlockSpec((1, _N_OUT), lambda i: (0, 0)),
        ],
        out_specs=pl.BlockSpec((_TILE, _N_OUT), lambda i: (i, 0)),
        out_shape=jax.ShapeDtypeStruct((b, _N_OUT), inputs.dtype),
    )(inputs, ac, b2d, r2d)
